# baseline (speedup 1.0000x reference)
import functools

import jax
import jax.numpy as jnp
from jax import lax
from jax.experimental import pallas as pl
from jax.experimental.pallas import tpu as pltpu
from jax.experimental.pallas import tpu_sc as plsc

N = 10000
E = 320000
KQ = 128

NC = 2
NS = 16
L = 16
NW = NC * NS
_SC_MESH = dict(core_axis_name="c", subcore_axis_name="s")


def _matmul_body(x_ref, w_ref, o_ref):
    o_ref[...] = jnp.dot(x_ref[...], w_ref[...],
                         preferred_element_type=jnp.float32)


def _matmul(x, w, block_rows=1000):
    n, din = x.shape
    dout = w.shape[1]
    grid = (n // block_rows,)
    return pl.pallas_call(
        _matmul_body,
        grid=grid,
        in_specs=[
            pl.BlockSpec((block_rows, din), lambda i: (i, 0)),
            pl.BlockSpec((din, dout), lambda i: (0, 0)),
        ],
        out_specs=pl.BlockSpec((block_rows, dout), lambda i: (i, 0)),
        out_shape=jax.ShapeDtypeStruct((n, dout), jnp.float32),
    )(x, w)


def _bn_body(num_ref, denp_ref, g_ref, b_ref, o_ref, *, residual_ref=None):
    den = jnp.sum(denp_ref[...], axis=0)
    half = N // 2
    x = jnp.concatenate([num_ref[0, :half, :], num_ref[1, :half, :]], axis=0)
    x = x / (den[:, None] + 1e-16)
    mu = jnp.mean(x, axis=0, keepdims=True)
    var = jnp.mean((x - mu) ** 2, axis=0, keepdims=True)
    y = g_ref[...] * (x - mu) / jnp.sqrt(var + 1e-5) + b_ref[...]
    if residual_ref is not None:
        y = y + residual_ref[...]
    o_ref[...] = jnp.maximum(y, 0.0)


def _bn_relu(num, denp, g, b, residual=None):
    nc, rpad, d = num.shape
    g2 = g.reshape(1, d)
    b2 = b.reshape(1, d)
    args = [num, denp, g2, b2]
    in_specs = [
        pl.BlockSpec((nc, rpad, d), lambda: (0, 0, 0)),
        pl.BlockSpec(denp.shape, lambda: (0, 0)),
        pl.BlockSpec((1, d), lambda: (0, 0)),
        pl.BlockSpec((1, d), lambda: (0, 0)),
    ]
    if residual is not None:
        body = lambda num_ref, denp_ref, g_ref, b_ref, r_ref, o_ref: _bn_body(
            num_ref, denp_ref, g_ref, b_ref, o_ref, residual_ref=r_ref)
        args.append(residual)
        in_specs.append(pl.BlockSpec((N, d), lambda: (0, 0)))
    else:
        body = _bn_body
    return pl.pallas_call(
        body,
        in_specs=in_specs,
        out_specs=pl.BlockSpec((N, d), lambda: (0, 0)),
        out_shape=jax.ShapeDtypeStruct((N, d), jnp.float32),
    )(*args)


_CB = 80
_EPW = E // NW


_GATHER_DNUMS = lax.GatherDimensionNumbers(
    offset_dims=(), collapsed_slice_dims=(0,), start_index_map=(0,))


def _lane_shuffle(x, perm):
    return lax.gather(x, perm[:, None], _GATHER_DNUMS, slice_sizes=(1,),
                      mode=lax.GatherScatterMode.PROMISE_IN_BOUNDS)


def _lane_sum(x):
    lane = lax.iota(jnp.int32, L)
    for sh in (1, 2, 4, 8):
        x = x + _lane_shuffle(x, lane ^ sh)
    return x


_NCH = _EPW // _CB


def _edge_scores_body(q_hbm, k_hbm, src_hbm, dst_hbm, ex_hbm, denp_hbm,
                      dstv, srcv, qrA, krA, qrB, krB, exv, den,
                      semA, semB):
    c = lax.axis_index("c")
    s = lax.axis_index("s")
    wid = s * NC + c

    zero16 = jnp.zeros((L,), jnp.float32)

    def zbody(i, carry):
        den[pl.ds(i * L, L)] = zero16
        return carry

    lax.fori_loop(0, N // L, zbody, 0)

    base0 = wid * _EPW
    scale = 1.0 / (KQ ** 0.5)

    pltpu.sync_copy(dst_hbm.at[pl.ds(base0, _EPW)], dstv)
    pltpu.sync_copy(src_hbm.at[pl.ds(base0, _EPW)], srcv)

    lane = lax.iota(jnp.int32, L)

    def issue(cidx, qr, kr, sem):
        cb = cidx * _CB
        pltpu.async_copy(q_hbm.at[dstv.at[pl.ds(cb, _CB)]], qr, sem)
        pltpu.async_copy(k_hbm.at[srcv.at[pl.ds(cb, _CB)]], kr, sem)

    def drain(qr, kr, sem):
        pltpu.make_async_copy(q_hbm.at[dstv.at[pl.ds(0, _CB)]], qr,
                              sem).wait()
        pltpu.make_async_copy(k_hbm.at[srcv.at[pl.ds(0, _CB)]], kr,
                              sem).wait()

    def compute(cidx, qr, kr):
        cb = cidx * _CB

        def grp(g, carry2):
            rb = g * L
            vecs = []
            for r in range(L):
                a = qr[rb + r, pl.ds(0, L)] * kr[rb + r, pl.ds(0, L)]
                for j in range(1, KQ // L):
                    a = a + (qr[rb + r, pl.ds(j * L, L)]
                             * kr[rb + r, pl.ds(j * L, L)])
                vecs.append(a)
            step = 1
            while len(vecs) > 1:
                m = (lane & step) == 0
                perm = lane ^ step
                nxt = []
                for p in range(0, len(vecs), 2):
                    a, b = vecs[p], vecs[p + 1]
                    nxt.append(jnp.where(m, a, b)
                               + _lane_shuffle(jnp.where(m, b, a), perm))
                vecs = nxt
                step *= 2
            ev = jnp.exp(vecs[0] * scale)
            exv[pl.ds(cb + rb, L)] = ev
            d16 = dstv[pl.ds(cb + rb, L)]
            plsc.addupdate_scatter(den, [d16], ev)
            return carry2

        lax.fori_loop(0, _CB // L, grp, 0)

    issue(0, qrA, krA, semA)

    def pair(i, carry):
        c0 = 2 * i
        issue(c0 + 1, qrB, krB, semB)
        drain(qrA, krA, semA)
        compute(c0, qrA, krA)
        issue(c0 + 2, qrA, krA, semA)
        drain(qrB, krB, semB)
        compute(c0 + 1, qrB, krB)
        return carry

    lax.fori_loop(0, (_NCH - 1) // 2, pair, 0)
    drain(qrA, krA, semA)
    compute(_NCH - 1, qrA, krA)

    pltpu.sync_copy(exv, ex_hbm.at[pl.ds(base0, _EPW)])
    pltpu.sync_copy(den, denp_hbm.at[wid])


def _edge_scores_sc(q, k, src, dst):
    f = pl.kernel(
        _edge_scores_body,
        out_type=[jax.ShapeDtypeStruct((E,), jnp.float32),
                  jax.ShapeDtypeStruct((NW, N), jnp.float32)],
        mesh=plsc.VectorSubcoreMesh(**_SC_MESH),
        scratch_types=[
            pltpu.VMEM((_EPW,), jnp.int32),
            pltpu.VMEM((_EPW,), jnp.int32),
            pltpu.VMEM((_CB, KQ), jnp.float32),
            pltpu.VMEM((_CB, KQ), jnp.float32),
            pltpu.VMEM((_CB, KQ), jnp.float32),
            pltpu.VMEM((_CB, KQ), jnp.float32),
            pltpu.VMEM((_EPW,), jnp.float32),
            pltpu.VMEM((N,), jnp.float32),
            pltpu.SemaphoreType.DMA,
            pltpu.SemaphoreType.DMA,
        ],
        compiler_params=pltpu.CompilerParams(needs_layout_passes=False),
    )
    return f(q, k, src, dst)


_HALF = N // NC
_RPS = 320
_RPAD = _RPS * NS
_SCAN = E // NS
_CB2 = 2000
_CE = 64
_LISTCAP = _SCAN + 4 * _CE


def _make_scatter_body(dv):
    def body(v_hbm, src_hbm, dst_hbm, ex_hbm, num_hbm,
             dstv, srcv, exv, lsrc, lex, ldst, vrA, vrB, zrow,
             acc, semA, semB, semSA, semSB):
        c = lax.axis_index("c")
        s = lax.axis_index("s")
        zero16 = jnp.zeros((L,), jnp.float32)
        izero16 = jnp.zeros((L,), jnp.int32)

        def zl(i, carry):
            lsrc[pl.ds(i * L, L)] = izero16
            ldst[pl.ds(i * L, L)] = izero16
            lex[pl.ds(i * L, L)] = zero16
            return carry

        lax.fori_loop(0, _LISTCAP // L, zl, 0)

        def zr(i, carry):
            for t in range(dv // L):
                zrow[i, pl.ds(t * L, L)] = zero16
            return carry

        lax.fori_loop(0, L, zr, 0)
        for m in range(_RPS // L):
            pltpu.sync_copy(zrow, acc.at[pl.ds(s * _RPS + m * L, L)])
        plsc.subcore_barrier()

        lo = c * _HALF
        base0 = s * _SCAN

        def scan_chunk(i, cur):
            base = base0 + i * _CB2
            pltpu.sync_copy(dst_hbm.at[pl.ds(base, _CB2)], dstv)
            pltpu.sync_copy(src_hbm.at[pl.ds(base, _CB2)], srcv)
            pltpu.sync_copy(ex_hbm.at[pl.ds(base, _CB2)], exv)

            def grp(g, cur2):
                gb = g * L
                d16 = dstv[pl.ds(gb, L)]
                ld16 = d16 - lo
                inb = (ld16 >= 0) & (ld16 < _HALF)
                cnt = plsc.all_reduce_population_count(inb)[0]
                plsc.store_compressed(lsrc.at[pl.ds(cur2, L)],
                                      srcv[pl.ds(gb, L)], mask=inb)
                plsc.store_compressed(lex.at[pl.ds(cur2, L)],
                                      exv[pl.ds(gb, L)], mask=inb)
                plsc.store_compressed(ldst.at[pl.ds(cur2, L)], ld16,
                                      mask=inb)
                return cur2 + cnt

            return lax.fori_loop(0, _CB2 // L, grp, cur)

        cnt_own = lax.fori_loop(0, _SCAN // _CB2, scan_chunk, 0)

        trips = (cnt_own + _CE - 1) // _CE
        pairs = trips // 2

        def pissue(j, vr, semg):
            pltpu.async_copy(v_hbm.at[lsrc.at[pl.ds(j * _CE, _CE)]], vr,
                             semg)

        def gdrain(vr, semg):
            pltpu.make_async_copy(v_hbm.at[lsrc.at[pl.ds(0, _CE)]], vr,
                                  semg).wait()

        def sissue(j, vr, sems):
            pltpu.async_copy(vr, acc.at[ldst.at[pl.ds(j * _CE, _CE)]],
                             sems, add=True)

        def sdrain(vr, sems):
            pltpu.make_async_copy(vr, acc.at[ldst.at[pl.ds(0, _CE)]],
                                  sems).wait()

        def scale(j, vr):
            jb = j * _CE

            def rbody(g2, carry2):
                rb2 = g2 * L
                ex16 = lex[pl.ds(jb + rb2, L)]
                for r in range(L):
                    sc = _lane_shuffle(ex16, jnp.full((L,), r, jnp.int32))
                    for t in range(dv // L):
                        vr[rb2 + r, pl.ds(t * L, L)] = (
                            vr[rb2 + r, pl.ds(t * L, L)] * sc)
                return carry2

            lax.fori_loop(0, _CE // L, rbody, 0)

        pissue(0, vrA, semA)

        def pair(i, carry):
            c0 = 2 * i
            gdrain(vrA, semA)

            @pl.when(c0 >= 1)
            def _():
                sdrain(vrB, semSB)

            pissue(c0 + 1, vrB, semB)
            scale(c0, vrA)
            sissue(c0, vrA, semSA)

            gdrain(vrB, semB)
            sdrain(vrA, semSA)
            pissue(c0 + 2, vrA, semA)
            scale(c0 + 1, vrB)
            sissue(c0 + 1, vrB, semSB)
            return carry

        lax.fori_loop(0, pairs, pair, 0)

        @pl.when(trips % 2 == 1)
        def _():
            gdrain(vrA, semA)

            @pl.when(pairs >= 1)
            def _():
                sdrain(vrB, semSB)

            pissue(2 * pairs + 1, vrB, semB)
            scale(2 * pairs, vrA)
            sissue(2 * pairs, vrA, semSA)

        @pl.when(trips % 2 == 0)
        def _():
            gdrain(vrA, semA)

        @pl.when(trips % 2 == 1)
        def _():
            gdrain(vrB, semB)
            sdrain(vrA, semSA)

        @pl.when((trips % 2 == 0) & (trips >= 1))
        def _():
            sdrain(vrB, semSB)

        plsc.subcore_barrier()
        pltpu.sync_copy(acc.at[pl.ds(s * _RPS, _RPS)],
                        num_hbm.at[c, pl.ds(s * _RPS, _RPS)])

    return body


def _edge_scatter_sc(v, src, dst, ex):
    dv = v.shape[1]
    f = pl.kernel(
        _make_scatter_body(dv),
        out_type=jax.ShapeDtypeStruct((NC, _RPAD, dv), jnp.float32),
        mesh=plsc.VectorSubcoreMesh(**_SC_MESH),
        scratch_types=[
            pltpu.VMEM((_CB2,), jnp.int32),
            pltpu.VMEM((_CB2,), jnp.int32),
            pltpu.VMEM((_CB2,), jnp.float32),
            pltpu.VMEM((_LISTCAP,), jnp.int32),
            pltpu.VMEM((_LISTCAP,), jnp.float32),
            pltpu.VMEM((_LISTCAP,), jnp.int32),
            pltpu.VMEM((_CE, dv), jnp.float32),
            pltpu.VMEM((_CE, dv), jnp.float32),
            pltpu.VMEM((L, dv), jnp.float32),
            pltpu.VMEM_SHARED((_RPAD, dv), jnp.float32),
            pltpu.SemaphoreType.DMA,
            pltpu.SemaphoreType.DMA,
            pltpu.SemaphoreType.DMA,
            pltpu.SemaphoreType.DMA,
        ],
        compiler_params=pltpu.CompilerParams(needs_layout_passes=False),
    )
    return f(v, src, dst, ex)


def _edge_stage(q, k, v, src, dst):
    ex, denp = _edge_scores_sc(q, k, src, dst)
    dv = v.shape[1]
    if dv > 128:
        num = jnp.concatenate(
            [_edge_scatter_sc(v[:, c:c + 128], src, dst, ex)
             for c in range(0, dv, 128)], axis=2)
    else:
        num = _edge_scatter_sc(v, src, dst, ex)
    return num, denp


def kernel(x0, edges, Wq1, Wk1, Wv1, g1, b1, Wq2, Wk2, Wv2, g2, b2):
    src = edges[0]
    dst = edges[1]

    W1 = jnp.concatenate([Wq1, Wk1, Wv1], axis=1)
    qkv1 = _matmul(x0, W1)
    q1, k1, v1 = qkv1[:, :KQ], qkv1[:, KQ:2 * KQ], qkv1[:, 2 * KQ:]
    num1, den1 = _edge_stage(q1, k1, v1, src, dst)
    x1 = _bn_relu(num1, den1, g1, b1)

    W2 = jnp.concatenate([Wq2, Wk2, Wv2], axis=1)
    qkv2 = _matmul(x1, W2)
    q2, k2, v2 = qkv2[:, :KQ], qkv2[:, KQ:2 * KQ], qkv2[:, 2 * KQ:]
    num2, den2 = _edge_stage(q2, k2, v2, src, dst)
    out = _bn_relu(num2, den2, g2, b2, residual=x0)
    return out


# speedup vs baseline: 1.1156x; 1.1156x over previous
import functools

import jax
import jax.numpy as jnp
from jax import lax
from jax.experimental import pallas as pl
from jax.experimental.pallas import tpu as pltpu
from jax.experimental.pallas import tpu_sc as plsc

N = 10000
E = 320000
KQ = 128

NC = 2
NS = 16
L = 16
NW = NC * NS
_SC_MESH = dict(core_axis_name="c", subcore_axis_name="s")


def _matmul_body(x_ref, w_ref, o_ref):
    o_ref[...] = jnp.dot(x_ref[...], w_ref[...],
                         preferred_element_type=jnp.float32)


def _matmul(x, w, block_rows=1000):
    n, din = x.shape
    dout = w.shape[1]
    grid = (n // block_rows,)
    return pl.pallas_call(
        _matmul_body,
        grid=grid,
        in_specs=[
            pl.BlockSpec((block_rows, din), lambda i: (i, 0)),
            pl.BlockSpec((din, dout), lambda i: (0, 0)),
        ],
        out_specs=pl.BlockSpec((block_rows, dout), lambda i: (i, 0)),
        out_shape=jax.ShapeDtypeStruct((n, dout), jnp.float32),
    )(x, w)


def _bn_body(num_ref, denp_ref, g_ref, b_ref, o_ref, *, residual_ref=None):
    den = jnp.sum(denp_ref[...], axis=0)
    half = N // 2
    x = jnp.concatenate([num_ref[0, :half, :], num_ref[1, :half, :]], axis=0)
    x = x / (den[:, None] + 1e-16)
    mu = jnp.mean(x, axis=0, keepdims=True)
    var = jnp.mean((x - mu) ** 2, axis=0, keepdims=True)
    y = g_ref[...] * (x - mu) / jnp.sqrt(var + 1e-5) + b_ref[...]
    if residual_ref is not None:
        y = y + residual_ref[...]
    o_ref[...] = jnp.maximum(y, 0.0)


def _bn_relu(num, denp, g, b, residual=None):
    nc, rpad, d = num.shape
    g2 = g.reshape(1, d)
    b2 = b.reshape(1, d)
    args = [num, denp, g2, b2]
    in_specs = [
        pl.BlockSpec((nc, rpad, d), lambda: (0, 0, 0)),
        pl.BlockSpec(denp.shape, lambda: (0, 0)),
        pl.BlockSpec((1, d), lambda: (0, 0)),
        pl.BlockSpec((1, d), lambda: (0, 0)),
    ]
    if residual is not None:
        body = lambda num_ref, denp_ref, g_ref, b_ref, r_ref, o_ref: _bn_body(
            num_ref, denp_ref, g_ref, b_ref, o_ref, residual_ref=r_ref)
        args.append(residual)
        in_specs.append(pl.BlockSpec((N, d), lambda: (0, 0)))
    else:
        body = _bn_body
    return pl.pallas_call(
        body,
        in_specs=in_specs,
        out_specs=pl.BlockSpec((N, d), lambda: (0, 0)),
        out_shape=jax.ShapeDtypeStruct((N, d), jnp.float32),
    )(*args)


_CB = 80
_EPW = E // NW


_GATHER_DNUMS = lax.GatherDimensionNumbers(
    offset_dims=(), collapsed_slice_dims=(0,), start_index_map=(0,))


def _lane_shuffle(x, perm):
    return lax.gather(x, perm[:, None], _GATHER_DNUMS, slice_sizes=(1,),
                      mode=lax.GatherScatterMode.PROMISE_IN_BOUNDS)


def _lane_sum(x):
    lane = lax.iota(jnp.int32, L)
    for sh in (1, 2, 4, 8):
        x = x + _lane_shuffle(x, lane ^ sh)
    return x


_NCH = _EPW // _CB


def _edge_scores_body(q_hbm, k_hbm, src_hbm, dst_hbm, ex_hbm, denp_hbm,
                      dstv, srcv, qrA, krA, qrB, krB, exv, den,
                      semA, semB):
    c = lax.axis_index("c")
    s = lax.axis_index("s")
    wid = s * NC + c

    zero16 = jnp.zeros((L,), jnp.float32)

    def zbody(i, carry):
        den[pl.ds(i * L, L)] = zero16
        return carry

    lax.fori_loop(0, N // L, zbody, 0)

    base0 = wid * _EPW
    scale = 1.0 / (KQ ** 0.5)

    pltpu.sync_copy(dst_hbm.at[pl.ds(base0, _EPW)], dstv)
    pltpu.sync_copy(src_hbm.at[pl.ds(base0, _EPW)], srcv)

    lane = lax.iota(jnp.int32, L)

    def issue(cidx, qr, kr, sem):
        cb = cidx * _CB
        pltpu.async_copy(q_hbm.at[dstv.at[pl.ds(cb, _CB)]], qr, sem)
        pltpu.async_copy(k_hbm.at[srcv.at[pl.ds(cb, _CB)]], kr, sem)

    def drain(qr, kr, sem):
        pltpu.make_async_copy(q_hbm.at[dstv.at[pl.ds(0, _CB)]], qr,
                              sem).wait()
        pltpu.make_async_copy(k_hbm.at[srcv.at[pl.ds(0, _CB)]], kr,
                              sem).wait()

    def compute(cidx, qr, kr):
        cb = cidx * _CB

        def grp(g, carry2):
            rb = g * L
            vecs = []
            for r in range(L):
                a = qr[rb + r, pl.ds(0, L)] * kr[rb + r, pl.ds(0, L)]
                for j in range(1, KQ // L):
                    a = a + (qr[rb + r, pl.ds(j * L, L)]
                             * kr[rb + r, pl.ds(j * L, L)])
                vecs.append(a)
            step = 1
            while len(vecs) > 1:
                m = (lane & step) == 0
                perm = lane ^ step
                nxt = []
                for p in range(0, len(vecs), 2):
                    a, b = vecs[p], vecs[p + 1]
                    nxt.append(jnp.where(m, a, b)
                               + _lane_shuffle(jnp.where(m, b, a), perm))
                vecs = nxt
                step *= 2
            ev = jnp.exp(vecs[0] * scale)
            exv[pl.ds(cb + rb, L)] = ev
            d16 = dstv[pl.ds(cb + rb, L)]
            plsc.addupdate_scatter(den, [d16], ev)
            return carry2

        lax.fori_loop(0, _CB // L, grp, 0)

    issue(0, qrA, krA, semA)

    def pair(i, carry):
        c0 = 2 * i
        issue(c0 + 1, qrB, krB, semB)
        drain(qrA, krA, semA)
        compute(c0, qrA, krA)
        issue(c0 + 2, qrA, krA, semA)
        drain(qrB, krB, semB)
        compute(c0 + 1, qrB, krB)
        return carry

    lax.fori_loop(0, (_NCH - 1) // 2, pair, 0)
    drain(qrA, krA, semA)
    compute(_NCH - 1, qrA, krA)

    pltpu.sync_copy(exv, ex_hbm.at[pl.ds(base0, _EPW)])
    pltpu.sync_copy(den, denp_hbm.at[wid])


def _edge_scores_sc(q, k, src, dst):
    f = pl.kernel(
        _edge_scores_body,
        out_type=[jax.ShapeDtypeStruct((E,), jnp.float32),
                  jax.ShapeDtypeStruct((NW, N), jnp.float32)],
        mesh=plsc.VectorSubcoreMesh(**_SC_MESH),
        scratch_types=[
            pltpu.VMEM((_EPW,), jnp.int32),
            pltpu.VMEM((_EPW,), jnp.int32),
            pltpu.VMEM((_CB, KQ), jnp.float32),
            pltpu.VMEM((_CB, KQ), jnp.float32),
            pltpu.VMEM((_CB, KQ), jnp.float32),
            pltpu.VMEM((_CB, KQ), jnp.float32),
            pltpu.VMEM((_EPW,), jnp.float32),
            pltpu.VMEM((N,), jnp.float32),
            pltpu.SemaphoreType.DMA,
            pltpu.SemaphoreType.DMA,
        ],
        compiler_params=pltpu.CompilerParams(needs_layout_passes=False),
    )
    return f(q, k, src, dst)


_HALF = N // NC
_RPS = 320
_RPAD = _RPS * NS
_SCAN = E // NS
_CB2 = 2000
_CE = 80
_LISTCAP = _SCAN + 4 * _CE


def _make_scatter_body(dv):
    def body(v_hbm, src_hbm, dst_hbm, ex_hbm, num_hbm,
             dstv, srcv, exv, lsrc, lex, ldst, vrA, vrB, zrow,
             acc, semA, semB):
        c = lax.axis_index("c")
        s = lax.axis_index("s")
        zero16 = jnp.zeros((L,), jnp.float32)
        izero16 = jnp.zeros((L,), jnp.int32)

        def zl(i, carry):
            lsrc[pl.ds(i * L, L)] = izero16
            ldst[pl.ds(i * L, L)] = izero16
            lex[pl.ds(i * L, L)] = zero16
            return carry

        lax.fori_loop(0, _LISTCAP // L, zl, 0)

        def zr(i, carry):
            for t in range(dv // L):
                zrow[i, pl.ds(t * L, L)] = zero16
            return carry

        lax.fori_loop(0, L, zr, 0)
        for m in range(_RPS // L):
            pltpu.sync_copy(zrow, acc.at[pl.ds(s * _RPS + m * L, L)])
        plsc.subcore_barrier()

        lo = c * _HALF
        base0 = s * _SCAN

        def scan_chunk(i, cur):
            base = base0 + i * _CB2
            pltpu.sync_copy(dst_hbm.at[pl.ds(base, _CB2)], dstv)
            pltpu.sync_copy(src_hbm.at[pl.ds(base, _CB2)], srcv)
            pltpu.sync_copy(ex_hbm.at[pl.ds(base, _CB2)], exv)

            def grp(g, cur2):
                gb = g * L
                d16 = dstv[pl.ds(gb, L)]
                ld16 = d16 - lo
                inb = (ld16 >= 0) & (ld16 < _HALF)
                cnt = plsc.all_reduce_population_count(inb)[0]
                plsc.store_compressed(lsrc.at[pl.ds(cur2, L)],
                                      srcv[pl.ds(gb, L)], mask=inb)
                plsc.store_compressed(lex.at[pl.ds(cur2, L)],
                                      exv[pl.ds(gb, L)], mask=inb)
                plsc.store_compressed(ldst.at[pl.ds(cur2, L)], ld16,
                                      mask=inb)
                return cur2 + cnt

            return lax.fori_loop(0, _CB2 // L, grp, cur)

        cnt_own = lax.fori_loop(0, _SCAN // _CB2, scan_chunk, 0)

        trips = (cnt_own + _CE - 1) // _CE
        pairs = trips // 2

        def pissue(j, vr, semg):
            pltpu.async_copy(v_hbm.at[lsrc.at[pl.ds(j * _CE, _CE)]], vr,
                             semg)

        def gdrain(vr, semg):
            pltpu.make_async_copy(v_hbm.at[lsrc.at[pl.ds(0, _CE)]], vr,
                                  semg).wait()

        def pwork(j, vr):
            jb = j * _CE

            def rbody(g2, carry2):
                rb2 = g2 * L
                ex16 = lex[pl.ds(jb + rb2, L)]
                for r in range(L):
                    sc = _lane_shuffle(ex16, jnp.full((L,), r, jnp.int32))
                    for t in range(dv // L):
                        vr[rb2 + r, pl.ds(t * L, L)] = (
                            vr[rb2 + r, pl.ds(t * L, L)] * sc)
                return carry2

            lax.fori_loop(0, _CE // L, rbody, 0)
            pltpu.sync_copy(vr, acc.at[ldst.at[pl.ds(jb, _CE)]], add=True)

        pissue(0, vrA, semA)

        def pair(i, carry):
            c0 = 2 * i
            pissue(c0 + 1, vrB, semB)
            gdrain(vrA, semA)
            pwork(c0, vrA)
            pissue(c0 + 2, vrA, semA)
            gdrain(vrB, semB)
            pwork(c0 + 1, vrB)
            return carry

        lax.fori_loop(0, pairs, pair, 0)
        gdrain(vrA, semA)

        @pl.when(trips % 2 == 1)
        def _():
            pwork(2 * pairs, vrA)

        plsc.subcore_barrier()
        pltpu.sync_copy(acc.at[pl.ds(s * _RPS, _RPS)],
                        num_hbm.at[c, pl.ds(s * _RPS, _RPS)])

    return body


def _edge_scatter_sc(v, src, dst, ex):
    dv = v.shape[1]
    f = pl.kernel(
        _make_scatter_body(dv),
        out_type=jax.ShapeDtypeStruct((NC, _RPAD, dv), jnp.float32),
        mesh=plsc.VectorSubcoreMesh(**_SC_MESH),
        scratch_types=[
            pltpu.VMEM((_CB2,), jnp.int32),
            pltpu.VMEM((_CB2,), jnp.int32),
            pltpu.VMEM((_CB2,), jnp.float32),
            pltpu.VMEM((_LISTCAP,), jnp.int32),
            pltpu.VMEM((_LISTCAP,), jnp.float32),
            pltpu.VMEM((_LISTCAP,), jnp.int32),
            pltpu.VMEM((_CE, dv), jnp.float32),
            pltpu.VMEM((_CE, dv), jnp.float32),
            pltpu.VMEM((L, dv), jnp.float32),
            pltpu.VMEM_SHARED((_RPAD, dv), jnp.float32),
            pltpu.SemaphoreType.DMA,
            pltpu.SemaphoreType.DMA,
        ],
        compiler_params=pltpu.CompilerParams(needs_layout_passes=False),
    )
    return f(v, src, dst, ex)


def _edge_stage(q, k, v, src, dst):
    ex, denp = _edge_scores_sc(q, k, src, dst)
    dv = v.shape[1]
    if dv > 128:
        num = jnp.concatenate(
            [_edge_scatter_sc(v[:, c:c + 128], src, dst, ex)
             for c in range(0, dv, 128)], axis=2)
    else:
        num = _edge_scatter_sc(v, src, dst, ex)
    return num, denp


def kernel(x0, edges, Wq1, Wk1, Wv1, g1, b1, Wq2, Wk2, Wv2, g2, b2):
    src = edges[0]
    dst = edges[1]

    W1 = jnp.concatenate([Wq1, Wk1, Wv1], axis=1)
    qkv1 = _matmul(x0, W1)
    q1, k1, v1 = qkv1[:, :KQ], qkv1[:, KQ:2 * KQ], qkv1[:, 2 * KQ:]
    num1, den1 = _edge_stage(q1, k1, v1, src, dst)
    x1 = _bn_relu(num1, den1, g1, b1)

    W2 = jnp.concatenate([Wq2, Wk2, Wv2], axis=1)
    qkv2 = _matmul(x1, W2)
    q2, k2, v2 = qkv2[:, :KQ], qkv2[:, KQ:2 * KQ], qkv2[:, 2 * KQ:]
    num2, den2 = _edge_stage(q2, k2, v2, src, dst)
    out = _bn_relu(num2, den2, g2, b2, residual=x0)
    return out


# speedup vs baseline: 1.1461x; 1.0273x over previous
import functools

import jax
import jax.numpy as jnp
from jax import lax
from jax.experimental import pallas as pl
from jax.experimental.pallas import tpu as pltpu
from jax.experimental.pallas import tpu_sc as plsc

N = 10000
E = 320000
KQ = 128

NC = 2
NS = 16
L = 16
NW = NC * NS
_SC_MESH = dict(core_axis_name="c", subcore_axis_name="s")


def _matmul_body(x_ref, w_ref, o_ref):
    o_ref[...] = jnp.dot(x_ref[...], w_ref[...],
                         preferred_element_type=jnp.float32)


def _matmul(x, w, block_rows=1000):
    n, din = x.shape
    dout = w.shape[1]
    grid = (n // block_rows,)
    return pl.pallas_call(
        _matmul_body,
        grid=grid,
        in_specs=[
            pl.BlockSpec((block_rows, din), lambda i: (i, 0)),
            pl.BlockSpec((din, dout), lambda i: (0, 0)),
        ],
        out_specs=pl.BlockSpec((block_rows, dout), lambda i: (i, 0)),
        out_shape=jax.ShapeDtypeStruct((n, dout), jnp.float32),
    )(x, w)


def _bn_body(num_ref, denp_ref, g_ref, b_ref, o_ref, *, residual_ref=None):
    den = jnp.sum(denp_ref[...], axis=0)
    half = N // 2
    x = jnp.concatenate([num_ref[0, :half, :], num_ref[1, :half, :]], axis=0)
    x = x / (den[:, None] + 1e-16)
    mu = jnp.mean(x, axis=0, keepdims=True)
    var = jnp.mean((x - mu) ** 2, axis=0, keepdims=True)
    y = g_ref[...] * (x - mu) / jnp.sqrt(var + 1e-5) + b_ref[...]
    if residual_ref is not None:
        y = y + residual_ref[...]
    o_ref[...] = jnp.maximum(y, 0.0)


def _bn_relu(num, denp, g, b, residual=None):
    nc, rpad, d = num.shape
    g2 = g.reshape(1, d)
    b2 = b.reshape(1, d)
    args = [num, denp, g2, b2]
    in_specs = [
        pl.BlockSpec((nc, rpad, d), lambda: (0, 0, 0)),
        pl.BlockSpec(denp.shape, lambda: (0, 0)),
        pl.BlockSpec((1, d), lambda: (0, 0)),
        pl.BlockSpec((1, d), lambda: (0, 0)),
    ]
    if residual is not None:
        body = lambda num_ref, denp_ref, g_ref, b_ref, r_ref, o_ref: _bn_body(
            num_ref, denp_ref, g_ref, b_ref, o_ref, residual_ref=r_ref)
        args.append(residual)
        in_specs.append(pl.BlockSpec((N, d), lambda: (0, 0)))
    else:
        body = _bn_body
    return pl.pallas_call(
        body,
        in_specs=in_specs,
        out_specs=pl.BlockSpec((N, d), lambda: (0, 0)),
        out_shape=jax.ShapeDtypeStruct((N, d), jnp.float32),
    )(*args)


_CB = 80
_EPW = E // NW


_GATHER_DNUMS = lax.GatherDimensionNumbers(
    offset_dims=(), collapsed_slice_dims=(0,), start_index_map=(0,))


def _lane_shuffle(x, perm):
    return lax.gather(x, perm[:, None], _GATHER_DNUMS, slice_sizes=(1,),
                      mode=lax.GatherScatterMode.PROMISE_IN_BOUNDS)


def _lane_sum(x):
    lane = lax.iota(jnp.int32, L)
    for sh in (1, 2, 4, 8):
        x = x + _lane_shuffle(x, lane ^ sh)
    return x


_NCH = _EPW // _CB


def _edge_scores_body(q_hbm, k_hbm, src_hbm, dst_hbm, ex_hbm, denp_hbm,
                      dstv, srcv, qrA, krA, qrB, krB, exv, den,
                      semA, semB):
    c = lax.axis_index("c")
    s = lax.axis_index("s")
    wid = s * NC + c

    zero16 = jnp.zeros((L,), jnp.float32)

    def zbody(i, carry):
        den[pl.ds(i * L, L)] = zero16
        return carry

    lax.fori_loop(0, N // L, zbody, 0)

    base0 = wid * _EPW
    scale = 1.0 / (KQ ** 0.5)

    pltpu.sync_copy(dst_hbm.at[pl.ds(base0, _EPW)], dstv)
    pltpu.sync_copy(src_hbm.at[pl.ds(base0, _EPW)], srcv)

    lane = lax.iota(jnp.int32, L)

    def issue(cidx, qr, kr, sem):
        cb = cidx * _CB
        pltpu.async_copy(q_hbm.at[dstv.at[pl.ds(cb, _CB)]], qr, sem)
        pltpu.async_copy(k_hbm.at[srcv.at[pl.ds(cb, _CB)]], kr, sem)

    def drain(qr, kr, sem):
        pltpu.make_async_copy(q_hbm.at[dstv.at[pl.ds(0, _CB)]], qr,
                              sem).wait()
        pltpu.make_async_copy(k_hbm.at[srcv.at[pl.ds(0, _CB)]], kr,
                              sem).wait()

    def compute(cidx, qr, kr):
        cb = cidx * _CB

        def grp(g, carry2):
            rb = g * L
            vecs = []
            for r in range(L):
                a = qr[rb + r, pl.ds(0, L)] * kr[rb + r, pl.ds(0, L)]
                for j in range(1, KQ // L):
                    a = a + (qr[rb + r, pl.ds(j * L, L)]
                             * kr[rb + r, pl.ds(j * L, L)])
                vecs.append(a)
            step = 1
            while len(vecs) > 1:
                m = (lane & step) == 0
                perm = lane ^ step
                nxt = []
                for p in range(0, len(vecs), 2):
                    a, b = vecs[p], vecs[p + 1]
                    nxt.append(jnp.where(m, a, b)
                               + _lane_shuffle(jnp.where(m, b, a), perm))
                vecs = nxt
                step *= 2
            ev = jnp.exp(vecs[0] * scale)
            exv[pl.ds(cb + rb, L)] = ev
            d16 = dstv[pl.ds(cb + rb, L)]
            plsc.addupdate_scatter(den, [d16], ev)
            return carry2

        lax.fori_loop(0, _CB // L, grp, 0)

    issue(0, qrA, krA, semA)

    def pair(i, carry):
        c0 = 2 * i
        issue(c0 + 1, qrB, krB, semB)
        drain(qrA, krA, semA)
        compute(c0, qrA, krA)
        issue(c0 + 2, qrA, krA, semA)
        drain(qrB, krB, semB)
        compute(c0 + 1, qrB, krB)
        return carry

    lax.fori_loop(0, (_NCH - 1) // 2, pair, 0)
    drain(qrA, krA, semA)
    compute(_NCH - 1, qrA, krA)

    pltpu.sync_copy(exv, ex_hbm.at[pl.ds(base0, _EPW)])
    pltpu.sync_copy(den, denp_hbm.at[wid])


def _edge_scores_sc(q, k, src, dst):
    f = pl.kernel(
        _edge_scores_body,
        out_type=[jax.ShapeDtypeStruct((E,), jnp.float32),
                  jax.ShapeDtypeStruct((NW, N), jnp.float32)],
        mesh=plsc.VectorSubcoreMesh(**_SC_MESH),
        scratch_types=[
            pltpu.VMEM((_EPW,), jnp.int32),
            pltpu.VMEM((_EPW,), jnp.int32),
            pltpu.VMEM((_CB, KQ), jnp.float32),
            pltpu.VMEM((_CB, KQ), jnp.float32),
            pltpu.VMEM((_CB, KQ), jnp.float32),
            pltpu.VMEM((_CB, KQ), jnp.float32),
            pltpu.VMEM((_EPW,), jnp.float32),
            pltpu.VMEM((N,), jnp.float32),
            pltpu.SemaphoreType.DMA,
            pltpu.SemaphoreType.DMA,
        ],
        compiler_params=pltpu.CompilerParams(needs_layout_passes=False),
    )
    return f(q, k, src, dst)


_HALF = N // NC
_RPS = 320
_RPAD = _RPS * NS
_SCAN = E // NS
_CB2 = 2000
_CE = 64
_LISTCAP = _SCAN + 4 * _CE


def _make_scatter_body(dv):
    def body(v_hbm, src_hbm, dst_hbm, ex_hbm, num_hbm,
             dstv, srcv, exv, lsrc, lex, ldst, idxbuf, vrA, vrB, zrow,
             acc, semA, semB):
        c = lax.axis_index("c")
        s = lax.axis_index("s")
        zero16 = jnp.zeros((L,), jnp.float32)
        izero16 = jnp.zeros((L,), jnp.int32)

        def zl(i, carry):
            lsrc[pl.ds(i * L, L)] = izero16
            ldst[pl.ds(i * L, L)] = izero16
            lex[pl.ds(i * L, L)] = zero16
            return carry

        lax.fori_loop(0, _LISTCAP // L, zl, 0)

        def zr(i, carry):
            for t in range(dv // L):
                zrow[i, pl.ds(t * L, L)] = zero16
            return carry

        lax.fori_loop(0, L, zr, 0)
        for m in range(_RPS // L):
            pltpu.sync_copy(zrow, acc.at[pl.ds(s * _RPS + m * L, L)])
        plsc.subcore_barrier()

        lo = c * _HALF
        base0 = s * _SCAN

        def scan_chunk(i, cur):
            base = base0 + i * _CB2
            pltpu.sync_copy(dst_hbm.at[pl.ds(base, _CB2)], dstv)
            pltpu.sync_copy(src_hbm.at[pl.ds(base, _CB2)], srcv)
            pltpu.sync_copy(ex_hbm.at[pl.ds(base, _CB2)], exv)

            def grp(g, cur2):
                gb = g * L
                d16 = dstv[pl.ds(gb, L)]
                ld16 = d16 - lo
                inb = (ld16 >= 0) & (ld16 < _HALF)
                cnt = plsc.all_reduce_population_count(inb)[0]
                plsc.store_compressed(lsrc.at[pl.ds(cur2, L)],
                                      srcv[pl.ds(gb, L)], mask=inb)
                plsc.store_compressed(lex.at[pl.ds(cur2, L)],
                                      exv[pl.ds(gb, L)], mask=inb)
                plsc.store_compressed(ldst.at[pl.ds(cur2, L)], ld16,
                                      mask=inb)
                return cur2 + cnt

            return lax.fori_loop(0, _CB2 // L, grp, cur)

        cnt_own = lax.fori_loop(0, _SCAN // _CB2, scan_chunk, 0)

        trips = (cnt_own + _CE - 1) // _CE
        pairs = trips // 2

        def pissue(j, vr, sem):
            pltpu.async_copy(v_hbm.at[lsrc.at[pl.ds(j * _CE, _CE)]], vr,
                             sem)

        def pdrain(vr, sem):
            pltpu.make_async_copy(v_hbm.at[lsrc.at[pl.ds(0, _CE)]], vr,
                                  sem).wait()

        def pwork(j, vr):
            jb = j * _CE
            for t in range(_CE // L):
                idxbuf[pl.ds(t * L, L)] = ldst[pl.ds(jb + t * L, L)]

            def rbody(g2, carry2):
                rb2 = g2 * L
                ex16 = lex[pl.ds(jb + rb2, L)]
                for r in range(L):
                    sc = _lane_shuffle(ex16, jnp.full((L,), r, jnp.int32))
                    for t in range(dv // L):
                        vr[rb2 + r, pl.ds(t * L, L)] = (
                            vr[rb2 + r, pl.ds(t * L, L)] * sc)
                return carry2

            lax.fori_loop(0, _CE // L, rbody, 0)
            pltpu.sync_copy(vr, acc.at[idxbuf], add=True)

        pissue(0, vrA, semA)

        def pair(i, carry):
            c0 = 2 * i
            pissue(c0 + 1, vrB, semB)
            pdrain(vrA, semA)
            pwork(c0, vrA)
            pissue(c0 + 2, vrA, semA)
            pdrain(vrB, semB)
            pwork(c0 + 1, vrB)
            return carry

        lax.fori_loop(0, pairs, pair, 0)
        pdrain(vrA, semA)

        @pl.when(trips % 2 == 1)
        def _():
            pwork(2 * pairs, vrA)

        plsc.subcore_barrier()
        pltpu.sync_copy(acc.at[pl.ds(s * _RPS, _RPS)],
                        num_hbm.at[c, pl.ds(s * _RPS, _RPS)])

    return body


def _edge_scatter_sc(v, src, dst, ex):
    dv = v.shape[1]
    f = pl.kernel(
        _make_scatter_body(dv),
        out_type=jax.ShapeDtypeStruct((NC, _RPAD, dv), jnp.float32),
        mesh=plsc.VectorSubcoreMesh(**_SC_MESH),
        scratch_types=[
            pltpu.VMEM((_CB2,), jnp.int32),
            pltpu.VMEM((_CB2,), jnp.int32),
            pltpu.VMEM((_CB2,), jnp.float32),
            pltpu.VMEM((_LISTCAP,), jnp.int32),
            pltpu.VMEM((_LISTCAP,), jnp.float32),
            pltpu.VMEM((_LISTCAP,), jnp.int32),
            pltpu.VMEM((_CE,), jnp.int32),
            pltpu.VMEM((_CE, dv), jnp.float32),
            pltpu.VMEM((_CE, dv), jnp.float32),
            pltpu.VMEM((L, dv), jnp.float32),
            pltpu.VMEM_SHARED((_RPAD, dv), jnp.float32),
            pltpu.SemaphoreType.DMA,
            pltpu.SemaphoreType.DMA,
        ],
        compiler_params=pltpu.CompilerParams(needs_layout_passes=False),
    )
    return f(v, src, dst, ex)


def _edge_stage(q, k, v, src, dst):
    ex, denp = _edge_scores_sc(q, k, src, dst)
    dv = v.shape[1]
    if dv > 128:
        num = jnp.concatenate(
            [_edge_scatter_sc(v[:, c:c + 128], src, dst, ex)
             for c in range(0, dv, 128)], axis=2)
    else:
        num = _edge_scatter_sc(v, src, dst, ex)
    return num, denp


def kernel(x0, edges, Wq1, Wk1, Wv1, g1, b1, Wq2, Wk2, Wv2, g2, b2):
    src = edges[0]
    dst = edges[1]

    W1 = jnp.concatenate([Wq1, Wk1, Wv1], axis=1)
    qkv1 = _matmul(x0, W1)
    q1, k1, v1 = qkv1[:, :KQ], qkv1[:, KQ:2 * KQ], qkv1[:, 2 * KQ:]
    num1, den1 = _edge_stage(q1, k1, v1, src, dst)
    x1 = _bn_relu(num1, den1, g1, b1)

    W2 = jnp.concatenate([Wq2, Wk2, Wv2], axis=1)
    qkv2 = _matmul(x1, W2)
    q2, k2, v2 = qkv2[:, :KQ], qkv2[:, KQ:2 * KQ], qkv2[:, 2 * KQ:]
    num2, den2 = _edge_stage(q2, k2, v2, src, dst)
    out = _bn_relu(num2, den2, g2, b2, residual=x0)
    return out
